# Initial kernel scaffold; baseline (speedup 1.0000x reference)
import jax
import jax.numpy as jnp
from jax.experimental import pallas as pl


def kernel(cls8, bbox8, cls16, bbox16, cls32, bbox32, origin_shapes):
    raise NotImplementedError("write your pallas kernel here")


# speedup vs baseline: 924.0241x; 924.0241x over previous
import numpy as np
import jax
import jax.numpy as jnp
from jax import lax
from jax.experimental import pallas as pl
from jax.experimental.pallas import tpu as pltpu

_B = 8
_N = 4200
_NPAD = 4224
_NOUT = 1024
_NOBJ = 1000
_THRES = 0.9
_IOU = 0.5


def _anchor_constants():
    cxs, cys, sts = [], [], []
    for stride in (8, 16, 32):
        hw = 320 // stride
        X, Y = np.meshgrid(np.arange(hw), np.arange(hw))
        ac = np.stack([X, Y], axis=-1).reshape(-1, 2) * stride
        ac = np.stack([ac, ac], axis=1).reshape(-1, 2).astype(np.float32)
        cxs.append(ac[:, 0])
        cys.append(ac[:, 1])
        sts.append(np.full(ac.shape[0], float(stride), np.float32))
    pad = np.zeros(_NPAD - _N, np.float32)
    cx = np.concatenate(cxs + [pad])[None, :]
    cy = np.concatenate(cys + [pad])[None, :]
    st = np.concatenate(sts + [pad])[None, :]
    return cx, cy, st


def _row_cumsum(x):
    ci = lax.broadcasted_iota(jnp.int32, x.shape, 1)
    sh = 1
    while sh < x.shape[1]:
        x = x + jnp.where(ci >= sh, jnp.roll(x, sh, axis=1), 0.0)
        sh *= 2
    return x


def _col_cumsum(x):
    ri = lax.broadcasted_iota(jnp.int32, x.shape, 0)
    for sh in (1, 2, 4):
        x = x + jnp.where(ri >= sh, jnp.roll(x, sh, axis=0), 0.0)
    return x


def _body(lg_ref, d0_ref, d1_ref, d2_ref, d3_ref, cx_ref, cy_ref, st_ref,
          org_ref, oy1, ox1, oy2, ox2, osc, state, kcnt):
    lg = lg_ref[...]
    s = jax.nn.sigmoid(lg)
    mask = s > _THRES
    mf = mask.astype(jnp.float32)

    ci = lax.broadcasted_iota(jnp.int32, (_B, _NPAD), 1)
    in8 = ci < 3200
    in16 = (ci >= 3200) & (ci < 4000)

    c8 = jnp.sum(jnp.where(in8, mf, 0.0), axis=1, keepdims=True)
    c16 = jnp.sum(jnp.where(in16, mf, 0.0), axis=1, keepdims=True)
    c32 = jnp.sum(mf, axis=1, keepdims=True) - c8 - c16
    t8 = jnp.sum(c8)
    t16 = jnp.sum(c16)

    off8 = _col_cumsum(c8) - c8
    off16 = t8 + _col_cumsum(c16) - c16
    off32 = t8 + t16 + _col_cumsum(c32) - c32
    col_off = jnp.where(in8, off8, jnp.where(in16, off16, off32))
    seg_base = jnp.where(in8, 0.0, jnp.where(in16, c8, c8 + c16))
    rank_incl = _row_cumsum(mf) - seg_base
    maskf = mask & (col_off + rank_incl <= float(_NOBJ))
    det = jnp.sum(maskf.astype(jnp.int32), axis=1, keepdims=True)

    org = org_ref[...]
    sy = jnp.sum(org[:, 0:1] * (1.0 / 320.0))
    sx = jnp.sum(org[:, 1:2] * (1.0 / 320.0))
    stv = st_ref[...]
    cx = cx_ref[...]
    cy = cy_ref[...]
    y1 = (cy - d1_ref[...] * stv) * sy
    x1 = (cx - d0_ref[...] * stv) * sx
    y2 = (cy + d3_ref[...] * stv) * sy
    x2 = (cx + d2_ref[...] * stv) * sx
    area = jnp.maximum(y2 - y1, 0.0) * jnp.maximum(x2 - x1, 0.0)

    state[...] = jnp.where(maskf, s, -jnp.inf)
    for b in range(_B):
        kcnt[b] = jnp.int32(0)

    def _inf_boxval(v):
        v = jnp.where(v == -1.0, jnp.inf, v)
        return jnp.where(v - 1.0 == -1.0, jnp.inf, v)

    def _cond(mmax):
        return mmax > _THRES

    def _step(mmax):
        st = state[...]
        m = jnp.max(st, axis=1, keepdims=True)
        idxm = jnp.min(jnp.where(st == m, ci, _NPAD), axis=1,
                       keepdims=True)
        act = m > _THRES
        eq = ci == idxm
        pick = lambda v: jnp.sum(jnp.where(eq, v, 0.0), axis=1, keepdims=True)
        y1i = pick(y1)
        x1i = pick(x1)
        y2i = pick(y2)
        x2i = pick(x2)
        ai = pick(area)
        yy1 = jnp.maximum(y1i, y1)
        xx1 = jnp.maximum(x1i, x1)
        yy2 = jnp.minimum(y2i, y2)
        xx2 = jnp.minimum(x2i, x2)
        inter = jnp.maximum(yy2 - yy1, 0.0) * jnp.maximum(xx2 - xx1, 0.0)
        union = ai + area - inter
        iou = jnp.where(union > 0.0, inter / jnp.maximum(union, 1e-12), 0.0)
        kill = ((iou > _IOU) | eq) & act
        st2 = jnp.where(kill, -jnp.inf, st)
        state[...] = st2
        for b in range(_B):
            @pl.when(m[b, 0] > _THRES)
            def _(b=b, y1i=y1i, x1i=x1i, y2i=y2i, x2i=x2i, m=m):
                k = kcnt[b]
                oy1[pl.ds(k, 1), b:b + 1] = jnp.reshape(_inf_boxval(y1i[b, 0]), (1, 1))
                ox1[pl.ds(k, 1), b:b + 1] = jnp.reshape(_inf_boxval(x1i[b, 0]), (1, 1))
                oy2[pl.ds(k, 1), b:b + 1] = jnp.reshape(_inf_boxval(y2i[b, 0]), (1, 1))
                ox2[pl.ds(k, 1), b:b + 1] = jnp.reshape(_inf_boxval(x2i[b, 0]), (1, 1))
                osc[pl.ds(k, 1), b:b + 1] = jnp.reshape(m[b, 0], (1, 1))
                kcnt[b] = k + 1
        return jnp.max(st2)

    lax.while_loop(_cond, _step, jnp.max(state[...]))

    ri = lax.broadcasted_iota(jnp.int32, (_NOUT, 1), 0)
    for b in range(_B):
        kb = kcnt[b]
        keep = ri < kb
        fake = ri < kb + (jnp.int32(_NOBJ) - det[b, 0])
        for ref in (oy1, ox1, oy2, ox2):
            ref[:, b:b + 1] = jnp.where(keep, ref[:, b:b + 1], jnp.inf)
        osc[:, b:b + 1] = jnp.where(
            keep, osc[:, b:b + 1], jnp.where(fake, jnp.inf, 0.0))


def kernel(cls8, bbox8, cls16, bbox16, cls32, bbox32, origin_shapes):
    cxc, cyc, stc = _anchor_constants()
    lg = jnp.concatenate([cls8[:, :, 0], cls16[:, :, 0], cls32[:, :, 0]], axis=1)
    lg = jnp.pad(lg, ((0, 0), (0, _NPAD - _N)), constant_values=-1e30)
    dch = []
    for i in range(4):
        d = jnp.concatenate([bbox8[:, :, i], bbox16[:, :, i], bbox32[:, :, i]], axis=1)
        dch.append(jnp.pad(d, ((0, 0), (0, _NPAD - _N))))
    outs = pl.pallas_call(
        _body,
        out_shape=[jax.ShapeDtypeStruct((_NOUT, _B), jnp.float32)] * 5,
        scratch_shapes=[
            pltpu.VMEM((_B, _NPAD), jnp.float32),
            pltpu.SMEM((_B,), jnp.int32),
        ],
    )(lg, dch[0], dch[1], dch[2], dch[3],
      jnp.asarray(cxc), jnp.asarray(cyc), jnp.asarray(stc), origin_shapes)
    y1, x1, y2, x2, sc = [o.T[:, :_NOBJ] for o in outs]
    cls_ch = jnp.zeros_like(sc)
    return jnp.stack([y1, x1, y2, x2, sc, cls_ch], axis=-1)
